# baseline (speedup 1.0000x reference)
import jax
import jax.numpy as jnp
from jax import lax
from jax.experimental import pallas as pl
from jax.experimental.pallas import tpu as pltpu
from jax.experimental.pallas import tpu_sc as plsc

N_NODES = 10000
K_SAMPLES = 4
D_IN = 256
D_OUT = 256
D_FLAT = K_SAMPLES * D_IN
CW = 128
NCHUNK = D_FLAT // CW
N_CORES = 2
N_TILES = 16
CHUNKS_PER_CORE = NCHUNK // N_CORES
E = 160000
EB = 128
NRANGE = 5
RN = 2000
NBLK_CAP = (E + EB - 1) // EB // N_TILES + 1
NBLK_P = NBLK_CAP + 3
ROWS_A = 640
ROWS_L = N_NODES - 15 * ROWS_A
WB_A = 128
WB_L = RN - 15 * WB_A
XOFF = RN


def _scale_block(gath, meta):
    for g in range(EB // 16):
        vv = lax.bitcast_convert_type(meta[2, pl.ds(g * 16, 16)],
                                      jnp.float32)
        for e in range(16):
            v = vv[e]
            r = g * 16 + e
            for q in range(CW // 16):
                gath[r, pl.ds(q * 16, 16)] = gath[r, pl.ds(q * 16, 16)] * v


def _sc_body(x_hbm, meta_hbm, cnt_hbm, zeros_hbm, out_hbm,
             g0, g1, m0, m1, cbuf, sg0, sg1, sm0, sm1, shr):
    cid = lax.axis_index("c")
    sid = lax.axis_index("s")
    gbuf = (g0, g1)
    mbuf = (m0, m1)
    sgs = (sg0, sg1)
    sms = (sm0, sm1)

    def chunk_iter(j, carry):
        c = cid * CHUNKS_PER_CORE + j
        srow0 = c * N_NODES

        @pl.when(sid < N_TILES - 1)
        def _():
            b0 = sid * ROWS_A
            pltpu.sync_copy(x_hbm.at[pl.ds(srow0 + b0, ROWS_A)],
                            shr.at[pl.ds(XOFF + b0, ROWS_A)])

        @pl.when(sid == N_TILES - 1)
        def _():
            b0 = (N_TILES - 1) * ROWS_A
            pltpu.sync_copy(x_hbm.at[pl.ds(srow0 + b0, ROWS_L)],
                            shr.at[pl.ds(XOFF + b0, ROWS_L)])

        plsc.subcore_barrier()

        def range_iter(r, carry2):
            mrow0 = (r * N_TILES + sid) * NBLK_P

            @pl.when(sid < N_TILES - 1)
            def _():
                pltpu.sync_copy(zeros_hbm, shr.at[pl.ds(sid * WB_A, WB_A)])

            @pl.when(sid == N_TILES - 1)
            def _():
                pltpu.sync_copy(zeros_hbm.at[pl.ds(0, WB_L)],
                                shr.at[pl.ds((N_TILES - 1) * WB_A, WB_L)])

            pltpu.sync_copy(cnt_hbm.at[r * N_TILES + sid], cbuf)
            nblk = cbuf[pl.ds(0, 16)][0]

            plsc.subcore_barrier()

            def start_gather(m, g, sg):
                pltpu.async_copy(shr.at[m.at[0]], g, sg)

            def wait_gather(m, g, sg):
                pltpu.make_async_copy(shr.at[m.at[0]], g, sg).wait()

            pltpu.sync_copy(meta_hbm.at[mrow0], m0)
            start_gather(m0, g0, sg0)
            pltpu.async_copy(meta_hbm.at[mrow0 + 1], m1, sm1)

            def pair_iter(t, _):
                for u in range(2):
                    b = 2 * t + u
                    cur, nxt = u, 1 - u
                    wait_gather(mbuf[cur], gbuf[cur], sgs[cur])
                    pltpu.make_async_copy(
                        meta_hbm.at[mrow0], mbuf[nxt], sms[nxt]).wait()
                    start_gather(mbuf[nxt], gbuf[nxt], sgs[nxt])
                    _scale_block(gbuf[cur], mbuf[cur])
                    pltpu.sync_copy(gbuf[cur], shr.at[mbuf[cur].at[1]],
                                    add=True)
                    pltpu.async_copy(meta_hbm.at[mrow0 + b + 2], mbuf[cur],
                                     sms[cur])
                return 0

            lax.fori_loop(0, nblk // 2, pair_iter, 0)
            wait_gather(m0, g0, sg0)
            pltpu.make_async_copy(meta_hbm.at[mrow0], m1, sm1).wait()

            plsc.subcore_barrier()

            orow0 = c * N_NODES + r * RN

            @pl.when(sid < N_TILES - 1)
            def _():
                b0 = sid * WB_A
                pltpu.sync_copy(shr.at[pl.ds(b0, WB_A)],
                                out_hbm.at[pl.ds(orow0 + b0, WB_A)])

            @pl.when(sid == N_TILES - 1)
            def _():
                b0 = (N_TILES - 1) * WB_A
                pltpu.sync_copy(shr.at[pl.ds(b0, WB_L)],
                                out_hbm.at[pl.ds(orow0 + b0, WB_L)])

            plsc.subcore_barrier()
            return carry2

        lax.fori_loop(0, NRANGE, range_iter, 0)
        return carry

    lax.fori_loop(0, CHUNKS_PER_CORE, chunk_iter, 0)


def _sc_spmm(xc, meta, counts):
    mesh = plsc.VectorSubcoreMesh(core_axis_name="c", subcore_axis_name="s",
                                  num_cores=N_CORES, num_subcores=N_TILES)
    f = pl.kernel(
        _sc_body,
        out_type=jax.ShapeDtypeStruct((NCHUNK * N_NODES, CW), jnp.float32),
        mesh=mesh,
        scratch_types=[
            pltpu.VMEM((EB, CW), jnp.float32),
            pltpu.VMEM((EB, CW), jnp.float32),
            pltpu.VMEM((3, EB), jnp.int32),
            pltpu.VMEM((3, EB), jnp.int32),
            pltpu.VMEM((16,), jnp.int32),
            pltpu.SemaphoreType.DMA,
            pltpu.SemaphoreType.DMA,
            pltpu.SemaphoreType.DMA,
            pltpu.SemaphoreType.DMA,
            pltpu.VMEM_SHARED((RN + N_NODES, CW), jnp.float32),
        ],
    )
    zeros = jnp.zeros((WB_A, CW), jnp.float32)
    return f(xc, meta, counts, zeros)


def _mm_body(a_ref, w_ref, o_ref):
    for k in range(K_SAMPLES):
        acc = jnp.dot(a_ref[2 * k], w_ref[0],
                      preferred_element_type=jnp.float32)
        acc = acc + jnp.dot(a_ref[2 * k + 1], w_ref[1],
                            preferred_element_type=jnp.float32)
        o_ref[:, k, :] = jnp.maximum(acc, 0.0)


def _matmul_relu(agg3, w3):
    NB = 2000
    grid = (N_NODES // NB,)
    return pl.pallas_call(
        _mm_body,
        grid=grid,
        in_specs=[
            pl.BlockSpec((NCHUNK, NB, CW), lambda nb: (0, nb, 0)),
            pl.BlockSpec((2, CW, D_OUT), lambda nb: (0, 0, 0)),
        ],
        out_specs=pl.BlockSpec((NB, K_SAMPLES, D_OUT), lambda nb: (nb, 0, 0)),
        out_shape=jax.ShapeDtypeStruct((N_NODES, K_SAMPLES, D_OUT), jnp.float32),
    )(agg3, w3)


def kernel(inputs, edge_index, edge_vals, W):
    x = inputs.reshape(N_NODES, D_FLAT)
    xc = x.reshape(N_NODES, NCHUNK, CW).transpose(1, 0, 2).reshape(
        NCHUNK * N_NODES, CW)
    dst = edge_index[0].astype(jnp.int32)
    src = edge_index[1].astype(jnp.int32)

    bucket = dst // RN
    order = jnp.argsort(bucket, stable=True)
    src_s = src[order]
    dst_s = dst[order]
    val_s = edge_vals[order]
    nper = jnp.bincount(bucket, length=NRANGE)
    starts = jnp.concatenate([jnp.zeros((1,), nper.dtype),
                              jnp.cumsum(nper)[:-1]])

    CAP_T = NBLK_P * EB
    lane = jnp.arange(N_TILES * CAP_T) // CAP_T
    slot = jnp.arange(N_TILES * CAP_T) % CAP_T
    k_of = slot * N_TILES + lane
    eid = starts[:, None] + k_of[None, :]
    valid = k_of[None, :] < nper[:, None]
    eid_c = jnp.clip(eid, 0, E - 1)
    m_src = jnp.where(valid, src_s[eid_c], 0) + XOFF
    m_dst = jnp.where(valid, dst_s[eid_c] - jnp.arange(NRANGE)[:, None] * RN,
                      0)
    m_val = jnp.where(valid, val_s[eid_c], 0.0)
    m_src = m_src.reshape(NRANGE, N_TILES, NBLK_P, EB)
    m_dst = m_dst.reshape(NRANGE, N_TILES, NBLK_P, EB)
    m_val = m_val.view(jnp.int32).reshape(NRANGE, N_TILES, NBLK_P, EB)
    meta = jnp.stack([m_src, m_dst, m_val], axis=3).reshape(
        NRANGE * N_TILES * NBLK_P, 3, EB)

    cnt_rt = (nper[:, None] + N_TILES - 1 - jnp.arange(N_TILES)) // N_TILES
    blk_rt = (cnt_rt + EB - 1) // EB
    blk_rt = (blk_rt + 1) // 2 * 2
    counts = jnp.broadcast_to(
        blk_rt.reshape(NRANGE * N_TILES, 1), (NRANGE * N_TILES, 16)
    ).astype(jnp.int32)

    agg = _sc_spmm(xc, meta, counts)
    agg3 = agg.reshape(NCHUNK, N_NODES, CW)
    return _matmul_relu(agg3, W.reshape(2, CW, D_OUT))
